# baseline (device time: 12585 ns/iter reference)
import functools

import jax
import jax.numpy as jnp
from jax import lax
from jax.experimental import pallas as pl
from jax.experimental.pallas import tpu as pltpu

N_DEV = 4
DC = 4


def kernel(t):
    m, n = t.shape
    rc = m // DC

    def body(x_ref, out_ref, stage_ref, comm_ref, diag_ref,
             send_sems, recv_sems, diag_send, diag_recv, diag_credit):
        my = lax.axis_index("i")
        nbrs = [(my + 1) % N_DEV, (my + 3) % N_DEV]
        diag = (my + 2) % N_DEV

        barrier_sem = pltpu.get_barrier_semaphore()
        for nbr in nbrs:
            pl.semaphore_signal(
                barrier_sem, inc=1,
                device_id=(nbr,), device_id_type=pl.DeviceIdType.MESH,
            )
        pl.semaphore_signal(
            diag_credit, inc=1,
            device_id=(diag,), device_id_type=pl.DeviceIdType.MESH,
        )

        stage_ref[:, :] = x_ref[:, :].astype(jnp.bfloat16)

        pl.semaphore_wait(diag_credit, 1)
        diag_rdmas = []
        for c in range(DC):
            rdma = pltpu.make_async_remote_copy(
                src_ref=stage_ref.at[pl.ds(c * rc, rc)],
                dst_ref=diag_ref.at[c],
                send_sem=diag_send.at[c],
                recv_sem=diag_recv.at[c],
                device_id=(diag,),
                device_id_type=pl.DeviceIdType.MESH,
            )
            rdma.start()
            diag_rdmas.append(rdma)

        pl.semaphore_wait(barrier_sem, 2)
        nbr_rdmas = []
        for i, nbr in enumerate(nbrs):
            rdma = pltpu.make_async_remote_copy(
                src_ref=stage_ref,
                dst_ref=comm_ref.at[i],
                send_sem=send_sems.at[i],
                recv_sem=recv_sems.at[i],
                device_id=(nbr,),
                device_id_type=pl.DeviceIdType.MESH,
            )
            rdma.start()
            nbr_rdmas.append(rdma)

        for rdma in nbr_rdmas:
            rdma.wait_recv()
        partial = (
            x_ref[:, :]
            + comm_ref[0, :, :].astype(jnp.float32)
            + comm_ref[1, :, :].astype(jnp.float32)
        )

        for c in range(DC):
            diag_rdmas[c].wait_recv()
            s = (partial[c * rc:(c + 1) * rc, :]
                 + diag_ref[c, :, :].astype(jnp.float32))
            r = jnp.maximum(s, 0.0)
            out_ref[pl.ds(c * rc, rc), :] = (
                jnp.tanh(s) * s * s + r * r * r
            ).astype(jnp.bfloat16)

        for rdma in nbr_rdmas:
            rdma.wait_send()
        for rdma in diag_rdmas:
            rdma.wait_send()

    return pl.pallas_call(
        body,
        out_shape=jax.ShapeDtypeStruct((m, n), jnp.bfloat16),
        in_specs=[pl.BlockSpec(memory_space=pltpu.VMEM)],
        out_specs=pl.BlockSpec(memory_space=pltpu.VMEM),
        scratch_shapes=[
            pltpu.VMEM((m, n), jnp.bfloat16),
            pltpu.VMEM((2, m, n), jnp.bfloat16),
            pltpu.VMEM((DC, rc, n), jnp.bfloat16),
            pltpu.SemaphoreType.DMA((2,)),
            pltpu.SemaphoreType.DMA((2,)),
            pltpu.SemaphoreType.DMA((DC,)),
            pltpu.SemaphoreType.DMA((DC,)),
            pltpu.SemaphoreType.REGULAR,
        ],
        compiler_params=pltpu.CompilerParams(collective_id=0),
    )(t)


# device time: 12566 ns/iter; 1.0015x vs baseline; 1.0015x over previous
import jax
import jax.numpy as jnp
from jax import lax
from jax.experimental import pallas as pl
from jax.experimental.pallas import tpu as pltpu

N_DEV = 4


def kernel(t):
    m, n = t.shape

    def body(x_ref, out_ref, stage_ref, comm_ref, send_sems, recv_sems):
        my = lax.axis_index("i")

        barrier_sem = pltpu.get_barrier_semaphore()
        for k in range(1, N_DEV):
            pl.semaphore_signal(
                barrier_sem, inc=1,
                device_id=((my + k) % N_DEV,),
                device_id_type=pl.DeviceIdType.MESH,
            )

        stage_ref[:, :] = x_ref[:, :].astype(jnp.bfloat16)

        pl.semaphore_wait(barrier_sem, N_DEV - 1)

        rdmas = {}
        for k in (2, 1, 3):
            rdma = pltpu.make_async_remote_copy(
                src_ref=stage_ref,
                dst_ref=comm_ref.at[k - 1],
                send_sem=send_sems.at[k - 1],
                recv_sem=recv_sems.at[k - 1],
                device_id=((my + k) % N_DEV,),
                device_id_type=pl.DeviceIdType.MESH,
            )
            rdma.start()
            rdmas[k] = rdma

        s = x_ref[:, :]
        for k in (1, 3, 2):
            rdmas[k].wait_recv()
            s = s + comm_ref[k - 1, :, :].astype(jnp.float32)
        r = jnp.maximum(s, 0.0)
        out_ref[:, :] = (jnp.tanh(s) * s * s + r * r * r).astype(jnp.bfloat16)

        for k in (1, 2, 3):
            rdmas[k].wait_send()

    return pl.pallas_call(
        body,
        out_shape=jax.ShapeDtypeStruct((m, n), jnp.bfloat16),
        in_specs=[pl.BlockSpec(memory_space=pltpu.VMEM)],
        out_specs=pl.BlockSpec(memory_space=pltpu.VMEM),
        scratch_shapes=[
            pltpu.VMEM((m, n), jnp.bfloat16),
            pltpu.VMEM((N_DEV - 1, m, n), jnp.bfloat16),
            pltpu.SemaphoreType.DMA((N_DEV - 1,)),
            pltpu.SemaphoreType.DMA((N_DEV - 1,)),
        ],
        compiler_params=pltpu.CompilerParams(collective_id=0),
    )(t)
